# Initial kernel scaffold; baseline (speedup 1.0000x reference)
import jax
import jax.numpy as jnp
from jax.experimental import pallas as pl


def kernel(x, edge_index, edge_weight, W1, b1, W2, b2):
    raise NotImplementedError("write your pallas kernel here")


# speedup vs baseline: 6.8296x; 6.8296x over previous
import functools

import jax
import jax.numpy as jnp
from jax import lax
from jax.experimental import pallas as pl
from jax.experimental.pallas import tpu as pltpu
from jax.experimental.pallas import tpu_sc as plsc

N = 10000
N_PAD = 10240
F = 256
D = 128
E = 160000
NC = 2
NS = 16
NW = NC * NS
B = 128
NB = 40
EPW = NB * B
RPT = N_PAD // NS

_MESH = plsc.VectorSubcoreMesh(core_axis_name="c", subcore_axis_name="s")


@functools.partial(
    pl.kernel,
    out_type=jax.ShapeDtypeStruct((NC, N_PAD), jnp.float32),
    mesh=_MESH,
    scratch_types=[
        pltpu.VMEM((NB, B), jnp.int32),
        pltpu.VMEM((NB, B), jnp.float32),
        pltpu.VMEM((RPT,), jnp.float32),
        pltpu.VMEM_SHARED((N_PAD,), jnp.float32),
    ],
)
def _deg_kernel(col_hbm, ew_hbm, deg_hbm, col_v, ew_v, zbuf, deg_sh):
    c = lax.axis_index("c")
    s = lax.axis_index("s")
    wid = c * NS + s
    pltpu.sync_copy(col_hbm.at[wid], col_v)
    pltpu.sync_copy(ew_hbm.at[wid], ew_v)

    def zb(i, carry):
        zbuf[pl.ds(i * 16, 16)] = jnp.zeros((16,), jnp.float32)
        return carry

    lax.fori_loop(0, RPT // 16, zb, 0)
    pltpu.sync_copy(zbuf, deg_sh.at[pl.ds(s * RPT, RPT)])
    plsc.subcore_barrier()

    def batch(b, carry):
        pltpu.sync_copy(ew_v.at[b], deg_sh.at[col_v.at[b]], add=True)
        return carry

    lax.fori_loop(0, NB, batch, 0)
    plsc.subcore_barrier()
    pltpu.sync_copy(deg_sh.at[pl.ds(s * RPT, RPT)],
                    deg_hbm.at[c, pl.ds(s * RPT, RPT)])


@functools.partial(
    pl.kernel,
    out_type=jax.ShapeDtypeStruct((NC, N_PAD, D), jnp.float32),
    mesh=_MESH,
    scratch_types=[
        pltpu.VMEM((NB, B), jnp.int32),
        pltpu.VMEM((NB, B), jnp.int32),
        pltpu.VMEM((EPW + 16,), jnp.float32),
        pltpu.VMEM((B, D), jnp.float32),
        pltpu.VMEM_SHARED((N_PAD, D), jnp.float32),
    ],
)
def _agg_kernel(src_hbm, row_hbm, col_hbm, ew_hbm, out_hbm,
                row_v, col_v, ew_v, rows_v, acc_sh):
    c = lax.axis_index("c")
    s = lax.axis_index("s")
    wid = c * NS + s
    pltpu.sync_copy(row_hbm.at[wid], row_v)
    pltpu.sync_copy(col_hbm.at[wid], col_v)
    pltpu.sync_copy(ew_hbm.at[wid], ew_v.at[pl.ds(0, EPW)])

    def zrow(i, carry):
        for f in range(D // 16):
            rows_v[i, pl.ds(f * 16, 16)] = jnp.zeros((16,), jnp.float32)
        return carry

    lax.fori_loop(0, B, zrow, 0)
    for k in range(RPT // B):
        pltpu.sync_copy(rows_v, acc_sh.at[pl.ds(s * RPT + k * B, B)])
    plsc.subcore_barrier()

    def batch(b, carry):
        pltpu.sync_copy(src_hbm.at[row_v.at[b]], rows_v)
        def edge(e, c2):
            w = ew_v[pl.ds(b * B + e, 16)][0]
            wv = jnp.full((16,), w, jnp.float32)
            for f in range(D // 16):
                sl = pl.ds(f * 16, 16)
                rows_v[e, sl] = rows_v[e, sl] * wv
            return c2

        lax.fori_loop(0, B, edge, 0)
        pltpu.sync_copy(rows_v, acc_sh.at[col_v.at[b]], add=True)
        return carry

    lax.fori_loop(0, NB, batch, 0)
    plsc.subcore_barrier()
    pltpu.sync_copy(acc_sh.at[pl.ds(s * RPT, RPT)],
                    out_hbm.at[c, pl.ds(s * RPT, RPT)])


BM = 1024
_GRID = N_PAD // BM


def _tc1_body(deg_ref, x_ref, w1_ref, dis_ref, src_ref):
    dis = lax.rsqrt(deg_ref[0] + deg_ref[1] + 1.0)
    dis_ref[...] = dis
    src_ref[...] = dis * jnp.dot(x_ref[...], w1_ref[...],
                                 preferred_element_type=jnp.float32)


_tc1 = pl.pallas_call(
    _tc1_body,
    grid=(_GRID,),
    in_specs=[
        pl.BlockSpec((2, BM, 1), lambda i: (0, i, 0)),
        pl.BlockSpec((BM, F), lambda i: (i, 0)),
        pl.BlockSpec((F, D), lambda i: (0, 0)),
    ],
    out_specs=[
        pl.BlockSpec((BM, 1), lambda i: (i, 0)),
        pl.BlockSpec((BM, D), lambda i: (i, 0)),
    ],
    out_shape=[
        jax.ShapeDtypeStruct((N_PAD, 1), jnp.float32),
        jax.ShapeDtypeStruct((N_PAD, D), jnp.float32),
    ],
)


def _tc2_body(p_ref, src1_ref, dis_ref, b1_ref, src2_ref):
    t = p_ref[0] + p_ref[1] + src1_ref[...]
    h = jnp.maximum(dis_ref[...] * t + b1_ref[...], 0.0)
    src2_ref[...] = dis_ref[...] * h


_tc2 = pl.pallas_call(
    _tc2_body,
    grid=(_GRID,),
    in_specs=[
        pl.BlockSpec((2, BM, D), lambda i: (0, i, 0)),
        pl.BlockSpec((BM, D), lambda i: (i, 0)),
        pl.BlockSpec((BM, 1), lambda i: (i, 0)),
        pl.BlockSpec((1, D), lambda i: (0, 0)),
    ],
    out_specs=pl.BlockSpec((BM, D), lambda i: (i, 0)),
    out_shape=jax.ShapeDtypeStruct((N_PAD, D), jnp.float32),
)


def _tc3_body(q_ref, src2_ref, dis_ref, w2_ref, b2_ref, out_ref):
    t = dis_ref[...] * (q_ref[0] + q_ref[1] + src2_ref[...])
    out_ref[...] = jnp.dot(t, w2_ref[...],
                           preferred_element_type=jnp.float32) + b2_ref[...]


_tc3 = pl.pallas_call(
    _tc3_body,
    grid=(_GRID,),
    in_specs=[
        pl.BlockSpec((2, BM, D), lambda i: (0, i, 0)),
        pl.BlockSpec((BM, D), lambda i: (i, 0)),
        pl.BlockSpec((BM, 1), lambda i: (i, 0)),
        pl.BlockSpec((D, F), lambda i: (0, 0)),
        pl.BlockSpec((1, F), lambda i: (0, 0)),
    ],
    out_specs=pl.BlockSpec((BM, F), lambda i: (i, 0)),
    out_shape=jax.ShapeDtypeStruct((N_PAD, F), jnp.float32),
)


def kernel(x, edge_index, edge_weight, W1, b1, W2, b2):
    row = edge_index[0].astype(jnp.int32)
    col = edge_index[1].astype(jnp.int32)
    ew = edge_weight.astype(jnp.float32)
    pad = NW * EPW - E
    rowp = jnp.pad(row, (0, pad)).reshape(NW, NB, B)
    colp = jnp.pad(col, (0, pad)).reshape(NW, NB, B)
    ewp = jnp.pad(ew, (0, pad)).reshape(NW, NB, B)
    ewf = ewp.reshape(NW, EPW)
    xpad = jnp.pad(x, ((0, N_PAD - N), (0, 0)))

    degp = _deg_kernel(colp, ewp)
    dis, src1 = _tc1(degp.reshape(NC, N_PAD, 1), xpad, W1)
    p = _agg_kernel(src1, rowp, colp, ewf)
    src2 = _tc2(p, src1, dis, b1.reshape(1, D))
    q = _agg_kernel(src2, rowp, colp, ewf)
    out = _tc3(q, src2, dis, W2, b2.reshape(1, F))
    return out[:N]


# speedup vs baseline: 9.1495x; 1.3397x over previous
import functools

import jax
import jax.numpy as jnp
from jax import lax
from jax.experimental import pallas as pl
from jax.experimental.pallas import tpu as pltpu
from jax.experimental.pallas import tpu_sc as plsc

N = 10000
N_PAD = 10240
F = 256
D = 128
E = 160000
NC = 2
NS = 16
NW = NC * NS
B = 64
NB = 80
EPW = NB * B
RPT = N_PAD // NS
NBUF = 2

_MESH = plsc.VectorSubcoreMesh(core_axis_name="c", subcore_axis_name="s")


@functools.partial(
    pl.kernel,
    out_type=jax.ShapeDtypeStruct((NC, N_PAD), jnp.float32),
    mesh=_MESH,
    scratch_types=[
        pltpu.VMEM((NB, B), jnp.int32),
        pltpu.VMEM((NB, B), jnp.float32),
        pltpu.VMEM((RPT,), jnp.float32),
        pltpu.VMEM_SHARED((N_PAD,), jnp.float32),
    ],
)
def _deg_kernel(col_hbm, ew_hbm, deg_hbm, col_v, ew_v, zbuf, deg_sh):
    c = lax.axis_index("c")
    s = lax.axis_index("s")
    wid = c * NS + s
    pltpu.sync_copy(col_hbm.at[wid], col_v)
    pltpu.sync_copy(ew_hbm.at[wid], ew_v)

    def zb(i, carry):
        zbuf[pl.ds(i * 16, 16)] = jnp.zeros((16,), jnp.float32)
        return carry

    lax.fori_loop(0, RPT // 16, zb, 0)
    pltpu.sync_copy(zbuf, deg_sh.at[pl.ds(s * RPT, RPT)])
    plsc.subcore_barrier()

    def batch(b, carry):
        pltpu.sync_copy(ew_v.at[b], deg_sh.at[col_v.at[b]], add=True)
        return carry

    lax.fori_loop(0, NB, batch, 0)
    plsc.subcore_barrier()
    pltpu.sync_copy(deg_sh.at[pl.ds(s * RPT, RPT)],
                    deg_hbm.at[c, pl.ds(s * RPT, RPT)])


@functools.partial(
    pl.kernel,
    out_type=jax.ShapeDtypeStruct((NC, N_PAD, D), jnp.float32),
    mesh=_MESH,
    scratch_types=[
        pltpu.VMEM((NB, B), jnp.int32),
        pltpu.VMEM((NB, B), jnp.int32),
        pltpu.VMEM((EPW,), jnp.float32),
        pltpu.VMEM((NBUF, B, D), jnp.float32),
        pltpu.VMEM_SHARED((N_PAD, D), jnp.float32),
        pltpu.SemaphoreType.DMA,
        pltpu.SemaphoreType.DMA,
        pltpu.SemaphoreType.DMA,
        pltpu.SemaphoreType.DMA,
    ],
)
def _agg_kernel(src_hbm, row_hbm, col_hbm, ew_hbm, out_hbm,
                row_v, col_v, ew_v, rows_v, acc_sh, *sems):
    gsem = sems[:NBUF]
    ssem = sems[NBUF:]
    c = lax.axis_index("c")
    s = lax.axis_index("s")
    wid = c * NS + s
    pltpu.sync_copy(row_hbm.at[wid], row_v)
    pltpu.sync_copy(col_hbm.at[wid], col_v)
    pltpu.sync_copy(ew_hbm.at[wid], ew_v)

    def zrow(i, carry):
        for f in range(D // 16):
            rows_v[0, i, pl.ds(f * 16, 16)] = jnp.zeros((16,), jnp.float32)
        return carry

    lax.fori_loop(0, B, zrow, 0)
    for k in range(RPT // B):
        pltpu.sync_copy(rows_v.at[0], acc_sh.at[pl.ds(s * RPT + k * B, B)])
    plsc.subcore_barrier()

    def _gather(b, j):
        pltpu.async_copy(src_hbm.at[row_v.at[b]], rows_v.at[j], gsem[j])

    def _scale(b, j):
        def group(g, carry):
            wg = ew_v[pl.ds((b * (B // 16) + g) * 16, 16)]
            for e16 in range(16):
                wv = jnp.full((16,), wg[e16], jnp.float32)
                e = g * 16 + e16
                for f in range(D // 16):
                    sl = pl.ds(f * 16, 16)
                    rows_v[j, e, sl] = rows_v[j, e, sl] * wv
            return carry

        lax.fori_loop(0, B // 16, group, 0)

    _gather(0, 0)
    _gather(1, 1)

    def rnd(r, carry):
        for j in range(NBUF):
            b = r * NBUF + j
            pltpu.make_async_copy(src_hbm.at[row_v.at[b]],
                                  rows_v.at[j], gsem[j]).wait()
            _scale(b, j)
            pltpu.async_copy(rows_v.at[j], acc_sh.at[col_v.at[b]], ssem[j],
                             add=True)
            jn = (j + 2) % NBUF
            bn = b + 2

            @pl.when(bn >= NBUF)
            def _():
                pltpu.make_async_copy(rows_v.at[jn],
                                      acc_sh.at[col_v.at[bn - NBUF]],
                                      ssem[jn]).wait()

            @pl.when(bn < NB)
            def _():
                _gather(bn, jn)
        return carry

    lax.fori_loop(0, NB // NBUF, rnd, 0)
    for j in range(2, NBUF):
        pltpu.make_async_copy(rows_v.at[j], acc_sh.at[col_v.at[NB - NBUF + j]],
                              ssem[j]).wait()
    plsc.subcore_barrier()
    pltpu.sync_copy(acc_sh.at[pl.ds(s * RPT, RPT)],
                    out_hbm.at[c, pl.ds(s * RPT, RPT)])


BM = 1024
_GRID = N_PAD // BM


def _tc1_body(deg_ref, x_ref, w1_ref, dis_ref, src_ref):
    dis = lax.rsqrt(deg_ref[0] + deg_ref[1] + 1.0)
    dis_ref[...] = dis
    src_ref[...] = dis * jnp.dot(x_ref[...], w1_ref[...],
                                 preferred_element_type=jnp.float32)


_tc1 = pl.pallas_call(
    _tc1_body,
    grid=(_GRID,),
    in_specs=[
        pl.BlockSpec((2, BM, 1), lambda i: (0, i, 0)),
        pl.BlockSpec((BM, F), lambda i: (i, 0)),
        pl.BlockSpec((F, D), lambda i: (0, 0)),
    ],
    out_specs=[
        pl.BlockSpec((BM, 1), lambda i: (i, 0)),
        pl.BlockSpec((BM, D), lambda i: (i, 0)),
    ],
    out_shape=[
        jax.ShapeDtypeStruct((N_PAD, 1), jnp.float32),
        jax.ShapeDtypeStruct((N_PAD, D), jnp.float32),
    ],
)


def _tc2_body(p_ref, src1_ref, dis_ref, b1_ref, src2_ref):
    t = p_ref[0] + p_ref[1] + src1_ref[...]
    h = jnp.maximum(dis_ref[...] * t + b1_ref[...], 0.0)
    src2_ref[...] = dis_ref[...] * h


_tc2 = pl.pallas_call(
    _tc2_body,
    grid=(_GRID,),
    in_specs=[
        pl.BlockSpec((2, BM, D), lambda i: (0, i, 0)),
        pl.BlockSpec((BM, D), lambda i: (i, 0)),
        pl.BlockSpec((BM, 1), lambda i: (i, 0)),
        pl.BlockSpec((1, D), lambda i: (0, 0)),
    ],
    out_specs=pl.BlockSpec((BM, D), lambda i: (i, 0)),
    out_shape=jax.ShapeDtypeStruct((N_PAD, D), jnp.float32),
)


def _tc3_body(q_ref, src2_ref, dis_ref, w2_ref, b2_ref, out_ref):
    t = dis_ref[...] * (q_ref[0] + q_ref[1] + src2_ref[...])
    out_ref[...] = jnp.dot(t, w2_ref[...],
                           preferred_element_type=jnp.float32) + b2_ref[...]


_tc3 = pl.pallas_call(
    _tc3_body,
    grid=(_GRID,),
    in_specs=[
        pl.BlockSpec((2, BM, D), lambda i: (0, i, 0)),
        pl.BlockSpec((BM, D), lambda i: (i, 0)),
        pl.BlockSpec((BM, 1), lambda i: (i, 0)),
        pl.BlockSpec((D, F), lambda i: (0, 0)),
        pl.BlockSpec((1, F), lambda i: (0, 0)),
    ],
    out_specs=pl.BlockSpec((BM, F), lambda i: (i, 0)),
    out_shape=jax.ShapeDtypeStruct((N_PAD, F), jnp.float32),
)


def kernel(x, edge_index, edge_weight, W1, b1, W2, b2):
    row = edge_index[0].astype(jnp.int32)
    col = edge_index[1].astype(jnp.int32)
    ew = edge_weight.astype(jnp.float32)
    pad = NW * EPW - E
    rowp = jnp.pad(row, (0, pad)).reshape(NW, NB, B)
    colp = jnp.pad(col, (0, pad)).reshape(NW, NB, B)
    ewp = jnp.pad(ew, (0, pad)).reshape(NW, NB, B)
    ewf = ewp.reshape(NW, EPW)
    xpad = jnp.pad(x, ((0, N_PAD - N), (0, 0)))

    degp = _deg_kernel(colp, ewp)
    dis, src1 = _tc1(degp.reshape(NC, N_PAD, 1), xpad, W1)
    p = _agg_kernel(src1, rowp, colp, ewf)
    src2 = _tc2(p, src1, dis, b1.reshape(1, D))
    q = _agg_kernel(src2, rowp, colp, ewf)
    out = _tc3(q, src2, dis, W2, b2.reshape(1, F))
    return out[:N]


# speedup vs baseline: 9.3615x; 1.0232x over previous
import functools

import jax
import jax.numpy as jnp
from jax import lax
from jax.experimental import pallas as pl
from jax.experimental.pallas import tpu as pltpu
from jax.experimental.pallas import tpu_sc as plsc

N = 10000
N_PAD = 10112
F = 256
D = 128
E = 160000
NC = 2
NS = 16
NW = NC * NS
B = 64
NB = 80
EPW = NB * B
RPT = N_PAD // NS
NBUF = 4
CH = 4
CHB = 8
NCH = NB // CHB

_MESH = plsc.VectorSubcoreMesh(core_axis_name="c", subcore_axis_name="s")


@functools.partial(
    pl.kernel,
    out_type=jax.ShapeDtypeStruct((NC * N_PAD,), jnp.float32),
    mesh=_MESH,
    scratch_types=[
        pltpu.VMEM((CH, B), jnp.int32),
        pltpu.VMEM((CH, B), jnp.float32),
        pltpu.VMEM((640,), jnp.float32),
        pltpu.VMEM_SHARED((N_PAD,), jnp.float32),
    ],
)
def _deg_kernel(col_hbm, ew_hbm, deg_hbm, col_c, ew_c, zbuf, deg_sh):
    c = lax.axis_index("c")
    s = lax.axis_index("s")
    wid = c * NS + s

    def zb(i, carry):
        zbuf[pl.ds(i * 16, 16)] = jnp.zeros((16,), jnp.float32)
        return carry

    lax.fori_loop(0, 640 // 16, zb, 0)
    pltpu.sync_copy(zbuf.at[pl.ds(0, RPT)], deg_sh.at[pl.ds(s * RPT, RPT)])
    plsc.subcore_barrier()

    def chunk(k, carry):
        pltpu.sync_copy(col_hbm.at[wid, pl.ds(k * CH, CH)], col_c)
        pltpu.sync_copy(ew_hbm.at[wid, pl.ds(k * CH, CH)], ew_c)
        for j in range(CH):
            pltpu.sync_copy(ew_c.at[j], deg_sh.at[col_c.at[j]], add=True)
        return carry

    lax.fori_loop(0, NB // CH, chunk, 0)
    plsc.subcore_barrier()
    pltpu.sync_copy(deg_sh.at[pl.ds(s * RPT, RPT)], zbuf.at[pl.ds(0, RPT)])
    pltpu.sync_copy(zbuf.at[pl.ds(0, RPT)],
                    deg_hbm.at[pl.ds(c * N_PAD + s * RPT, RPT)])


@functools.partial(
    pl.kernel,
    out_type=jax.ShapeDtypeStruct((NC, N_PAD, D), jnp.float32),
    mesh=_MESH,
    scratch_types=[
        pltpu.VMEM((2, CHB, B), jnp.int32),
        pltpu.VMEM((2, CHB, B), jnp.int32),
        pltpu.VMEM((2, CHB, B), jnp.float32),
        pltpu.VMEM((NBUF, B, D), jnp.float32),
        pltpu.VMEM_SHARED((N_PAD, D), jnp.float32),
        pltpu.SemaphoreType.DMA,
        pltpu.SemaphoreType.DMA,
        pltpu.SemaphoreType.DMA,
        pltpu.SemaphoreType.DMA,
        pltpu.SemaphoreType.DMA,
        pltpu.SemaphoreType.DMA,
        pltpu.SemaphoreType.DMA,
        pltpu.SemaphoreType.DMA,
    ],
)
def _agg_kernel(src_hbm, row_hbm, col_hbm, ew_hbm, out_hbm,
                row_c, col_c, ew_c, rows_v, acc_sh, *sems):
    gsem = sems[:NBUF]
    ssem = sems[NBUF:]
    c = lax.axis_index("c")
    s = lax.axis_index("s")
    wid = c * NS + s

    def _stage(k, slot):
        pltpu.sync_copy(row_hbm.at[wid, pl.ds(k * CHB, CHB)], row_c.at[slot])
        pltpu.sync_copy(col_hbm.at[wid, pl.ds(k * CHB, CHB)], col_c.at[slot])
        pltpu.sync_copy(ew_hbm.at[wid, pl.ds(k * CHB, CHB)], ew_c.at[slot])

    def zrow(i, carry):
        for f in range(D // 16):
            rows_v[0, i, pl.ds(f * 16, 16)] = jnp.zeros((16,), jnp.float32)
        return carry

    lax.fori_loop(0, B, zrow, 0)
    for k in range(RPT // B):
        pltpu.sync_copy(rows_v.at[0], acc_sh.at[pl.ds(s * RPT + k * B, B)])
    if RPT % B:
        pltpu.sync_copy(
            rows_v.at[0, pl.ds(0, RPT % B)],
            acc_sh.at[pl.ds(s * RPT + RPT - RPT % B, RPT % B)])
    plsc.subcore_barrier()

    def _scale(slot, jb, j):
        def group(g, carry):
            wg = ew_c[slot, jb, pl.ds(g * 16, 16)]
            for e16 in range(16):
                wv = jnp.full((16,), wg[e16], jnp.float32)
                e = g * 16 + e16
                for f in range(D // 16):
                    sl = pl.ds(f * 16, 16)
                    rows_v[j, e, sl] = rows_v[j, e, sl] * wv
            return carry

        lax.fori_loop(0, B // 16, group, 0)

    _stage(0, 0)
    _stage(1, 1)
    pltpu.async_copy(src_hbm.at[row_c.at[0, 0]], rows_v.at[0], gsem[0])
    pltpu.async_copy(src_hbm.at[row_c.at[0, 1]], rows_v.at[1], gsem[1])

    def chunk_loop(k, carry):
        slot = lax.rem(k, 2)
        nslot = lax.rem(k + 1, 2)
        for jb in range(CHB):
            b = k * CHB + jb
            j = jb % NBUF
            jn = (jb + 2) % NBUF
            pltpu.make_async_copy(src_hbm.at[row_c.at[slot, jb]],
                                  rows_v.at[j], gsem[j]).wait()
            _scale(slot, jb, j)
            pltpu.async_copy(rows_v.at[j], acc_sh.at[col_c.at[slot, jb]],
                             ssem[j], add=True)

            @pl.when(b >= 2)
            def _():
                pltpu.make_async_copy(rows_v.at[jn],
                                      acc_sh.at[col_c.at[slot, jb]],
                                      ssem[jn]).wait()

            if jb < CHB - 2:
                pltpu.async_copy(src_hbm.at[row_c.at[slot, jb + 2]],
                                 rows_v.at[jn], gsem[jn])
            else:
                @pl.when(k + 1 < NCH)
                def _():
                    pltpu.async_copy(src_hbm.at[row_c.at[nslot, jb - (CHB - 2)]],
                                     rows_v.at[jn], gsem[jn])

            if jb == 1:
                @pl.when((k >= 1) & (k + 1 < NCH))
                def _():
                    _stage(k + 1, nslot)
        return carry

    lax.fori_loop(0, NCH, chunk_loop, 0)
    for j in range(2, NBUF):
        pltpu.make_async_copy(rows_v.at[j], acc_sh.at[col_c.at[0, 0]],
                              ssem[j]).wait()
    plsc.subcore_barrier()
    pltpu.sync_copy(acc_sh.at[pl.ds(s * RPT, RPT)],
                    out_hbm.at[c, pl.ds(s * RPT, RPT)])


BM = RPT
_GRID = N_PAD // BM


def _tc1_body(deg_ref, x_ref, w1_ref, dis_ref, src_ref):
    dis = lax.rsqrt(deg_ref[0] + deg_ref[1] + 1.0)
    dis_ref[...] = dis
    src_ref[...] = dis * jnp.dot(x_ref[...], w1_ref[...],
                                 preferred_element_type=jnp.float32)


_tc1 = pl.pallas_call(
    _tc1_body,
    grid=(_GRID,),
    in_specs=[
        pl.BlockSpec((2, BM, 1), lambda i: (0, i, 0)),
        pl.BlockSpec((BM, F), lambda i: (i, 0)),
        pl.BlockSpec((F, D), lambda i: (0, 0)),
    ],
    out_specs=[
        pl.BlockSpec((BM, 1), lambda i: (i, 0)),
        pl.BlockSpec((BM, D), lambda i: (i, 0)),
    ],
    out_shape=[
        jax.ShapeDtypeStruct((N_PAD, 1), jnp.float32),
        jax.ShapeDtypeStruct((N_PAD, D), jnp.float32),
    ],
)


def _tc2_body(p_ref, src1_ref, dis_ref, b1_ref, src2_ref):
    t = p_ref[0] + p_ref[1] + src1_ref[...]
    h = jnp.maximum(dis_ref[...] * t + b1_ref[...], 0.0)
    src2_ref[...] = dis_ref[...] * h


_tc2 = pl.pallas_call(
    _tc2_body,
    grid=(_GRID,),
    in_specs=[
        pl.BlockSpec((2, BM, D), lambda i: (0, i, 0)),
        pl.BlockSpec((BM, D), lambda i: (i, 0)),
        pl.BlockSpec((BM, 1), lambda i: (i, 0)),
        pl.BlockSpec((1, D), lambda i: (0, 0)),
    ],
    out_specs=pl.BlockSpec((BM, D), lambda i: (i, 0)),
    out_shape=jax.ShapeDtypeStruct((N_PAD, D), jnp.float32),
)


def _tc3_body(q_ref, src2_ref, dis_ref, w2_ref, b2_ref, out_ref):
    t = dis_ref[...] * (q_ref[0] + q_ref[1] + src2_ref[...])
    out_ref[...] = jnp.dot(t, w2_ref[...],
                           preferred_element_type=jnp.float32) + b2_ref[...]


_tc3 = pl.pallas_call(
    _tc3_body,
    grid=(_GRID,),
    in_specs=[
        pl.BlockSpec((2, BM, D), lambda i: (0, i, 0)),
        pl.BlockSpec((BM, D), lambda i: (i, 0)),
        pl.BlockSpec((BM, 1), lambda i: (i, 0)),
        pl.BlockSpec((D, F), lambda i: (0, 0)),
        pl.BlockSpec((1, F), lambda i: (0, 0)),
    ],
    out_specs=pl.BlockSpec((BM, F), lambda i: (i, 0)),
    out_shape=jax.ShapeDtypeStruct((N_PAD, F), jnp.float32),
)


def kernel(x, edge_index, edge_weight, W1, b1, W2, b2):
    row = edge_index[0].astype(jnp.int32)
    col = edge_index[1].astype(jnp.int32)
    ew = edge_weight.astype(jnp.float32)
    pad = NW * EPW - E
    rowp = jnp.pad(row, (0, pad)).reshape(NW, NB, B)
    colp = jnp.pad(col, (0, pad)).reshape(NW, NB, B)
    ewp = jnp.pad(ew, (0, pad)).reshape(NW, NB, B)
    xpad = jnp.pad(x, ((0, N_PAD - N), (0, 0)))

    degp = _deg_kernel(colp, ewp)
    dis, src1 = _tc1(degp.reshape(NC, N_PAD, 1), xpad, W1)
    p = _agg_kernel(src1, rowp, colp, ewp)
    src2 = _tc2(p, src1, dis, b1.reshape(1, D))
    q = _agg_kernel(src2, rowp, colp, ewp)
    out = _tc3(q, src2, dis, W2, b2.reshape(1, F))
    return out[:N]


# speedup vs baseline: 10.5691x; 1.1290x over previous
import functools

import jax
import jax.numpy as jnp
from jax import lax
from jax.experimental import pallas as pl
from jax.experimental.pallas import tpu as pltpu
from jax.experimental.pallas import tpu_sc as plsc

N = 10000
N_PAD = 10112
F = 256
D = 128
E = 160000
NC = 2
NS = 16
NW = NC * NS
B = 64
RPT = N_PAD // NS
NBUF = 4
CHB = 8
SLOW_C = 1
NCHF = 15
NCHS = 5
NB = NCHF * CHB
EPW_F = NCHF * CHB * B
EPW_S = NCHS * CHB * B
NB_DEG = 80
CH = 16

_MESH = plsc.VectorSubcoreMesh(core_axis_name="c", subcore_axis_name="s")


@functools.partial(
    pl.kernel,
    out_type=jax.ShapeDtypeStruct((NC * N_PAD,), jnp.float32),
    mesh=_MESH,
    scratch_types=[
        pltpu.VMEM((CH, B), jnp.int32),
        pltpu.VMEM((CH, B), jnp.float32),
        pltpu.VMEM((640,), jnp.float32),
        pltpu.VMEM_SHARED((N_PAD,), jnp.float32),
    ],
)
def _deg_kernel(col_hbm, ew_hbm, deg_hbm, col_c, ew_c, zbuf, deg_sh):
    c = lax.axis_index("c")
    s = lax.axis_index("s")
    wid = c * NS + s

    def zb(i, carry):
        zbuf[pl.ds(i * 16, 16)] = jnp.zeros((16,), jnp.float32)
        return carry

    lax.fori_loop(0, 640 // 16, zb, 0)
    pltpu.sync_copy(zbuf.at[pl.ds(0, RPT)], deg_sh.at[pl.ds(s * RPT, RPT)])
    plsc.subcore_barrier()

    def chunk(k, carry):
        pltpu.sync_copy(col_hbm.at[wid, pl.ds(k * CH, CH)], col_c)
        pltpu.sync_copy(ew_hbm.at[wid, pl.ds(k * CH, CH)], ew_c)
        for j in range(CH):
            pltpu.sync_copy(ew_c.at[j], deg_sh.at[col_c.at[j]], add=True)
        return carry

    lax.fori_loop(0, NB_DEG // CH, chunk, 0)
    plsc.subcore_barrier()
    pltpu.sync_copy(deg_sh.at[pl.ds(s * RPT, RPT)], zbuf.at[pl.ds(0, RPT)])
    pltpu.sync_copy(zbuf.at[pl.ds(0, RPT)],
                    deg_hbm.at[pl.ds(c * N_PAD + s * RPT, RPT)])


@functools.partial(
    pl.kernel,
    out_type=jax.ShapeDtypeStruct((NC, N_PAD, D), jnp.float32),
    mesh=_MESH,
    scratch_types=[
        pltpu.VMEM((2, CHB, B), jnp.int32),
        pltpu.VMEM((2, CHB, B), jnp.int32),
        pltpu.VMEM((2, CHB, B), jnp.float32),
        pltpu.VMEM((NBUF, B, D), jnp.float32),
        pltpu.VMEM_SHARED((N_PAD, D), jnp.float32),
        pltpu.SemaphoreType.DMA,
        pltpu.SemaphoreType.DMA,
        pltpu.SemaphoreType.DMA,
        pltpu.SemaphoreType.DMA,
        pltpu.SemaphoreType.DMA,
        pltpu.SemaphoreType.DMA,
        pltpu.SemaphoreType.DMA,
        pltpu.SemaphoreType.DMA,
    ],
)
def _agg_kernel(src_hbm, row_hbm, col_hbm, ew_hbm, out_hbm,
                row_c, col_c, ew_c, rows_v, acc_sh, *sems):
    gsem = sems[:NBUF]
    ssem = sems[NBUF:]
    c = lax.axis_index("c")
    s = lax.axis_index("s")
    wid = c * NS + s

    def _stage(k, slot):
        pltpu.sync_copy(row_hbm.at[wid, pl.ds(k * CHB, CHB)], row_c.at[slot])
        pltpu.sync_copy(col_hbm.at[wid, pl.ds(k * CHB, CHB)], col_c.at[slot])
        pltpu.sync_copy(ew_hbm.at[wid, pl.ds(k * CHB, CHB)], ew_c.at[slot])

    def zrow(i, carry):
        for f in range(D // 16):
            rows_v[0, i, pl.ds(f * 16, 16)] = jnp.zeros((16,), jnp.float32)
        return carry

    lax.fori_loop(0, B, zrow, 0)
    for k in range(RPT // B):
        pltpu.sync_copy(rows_v.at[0], acc_sh.at[pl.ds(s * RPT + k * B, B)])
    if RPT % B:
        pltpu.sync_copy(
            rows_v.at[0, pl.ds(0, RPT % B)],
            acc_sh.at[pl.ds(s * RPT + RPT - RPT % B, RPT % B)])
    plsc.subcore_barrier()

    def _scale(slot, jb, j):
        def group(g, carry):
            wg = ew_c[slot, jb, pl.ds(g * 16, 16)]
            for e16 in range(16):
                wv = jnp.full((16,), wg[e16], jnp.float32)
                e = g * 16 + e16
                for f in range(D // 16):
                    sl = pl.ds(f * 16, 16)
                    rows_v[j, e, sl] = rows_v[j, e, sl] * wv
            return carry

        lax.fori_loop(0, B // 16, group, 0)

    _stage(0, 0)
    _stage(1, 1)
    pltpu.async_copy(src_hbm.at[row_c.at[0, 0]], rows_v.at[0], gsem[0])
    pltpu.async_copy(src_hbm.at[row_c.at[0, 1]], rows_v.at[1], gsem[1])

    nch = jnp.where(c == SLOW_C, NCHS, NCHF)

    def chunk_loop(k, carry):
        slot = lax.rem(k, 2)
        nslot = lax.rem(k + 1, 2)
        for jb in range(CHB):
            b = k * CHB + jb
            j = jb % NBUF
            jn = (jb + 2) % NBUF
            pltpu.make_async_copy(src_hbm.at[row_c.at[slot, jb]],
                                  rows_v.at[j], gsem[j]).wait()
            _scale(slot, jb, j)
            pltpu.async_copy(rows_v.at[j], acc_sh.at[col_c.at[slot, jb]],
                             ssem[j], add=True)

            @pl.when(b >= 2)
            def _():
                pltpu.make_async_copy(rows_v.at[jn],
                                      acc_sh.at[col_c.at[slot, jb]],
                                      ssem[jn]).wait()

            if jb < CHB - 2:
                pltpu.async_copy(src_hbm.at[row_c.at[slot, jb + 2]],
                                 rows_v.at[jn], gsem[jn])
            else:
                @pl.when(k + 1 < nch)
                def _():
                    pltpu.async_copy(src_hbm.at[row_c.at[nslot, jb - (CHB - 2)]],
                                     rows_v.at[jn], gsem[jn])

            if jb == 1:
                @pl.when((k >= 1) & (k + 1 < nch))
                def _():
                    _stage(k + 1, nslot)
        return carry

    lax.fori_loop(0, nch, chunk_loop, 0)
    for j in range(2, NBUF):
        pltpu.make_async_copy(rows_v.at[j], acc_sh.at[col_c.at[0, 0]],
                              ssem[j]).wait()
    plsc.subcore_barrier()
    pltpu.sync_copy(acc_sh.at[pl.ds(s * RPT, RPT)],
                    out_hbm.at[c, pl.ds(s * RPT, RPT)])


BM = RPT
_GRID = N_PAD // BM


def _tc1_body(deg_ref, x_ref, w1_ref, dis_ref, src_ref):
    dis = lax.rsqrt(deg_ref[0] + deg_ref[1] + 1.0)
    dis_ref[...] = dis
    src_ref[...] = dis * jnp.dot(x_ref[...], w1_ref[...],
                                 preferred_element_type=jnp.float32)


_tc1 = pl.pallas_call(
    _tc1_body,
    grid=(_GRID,),
    in_specs=[
        pl.BlockSpec((2, BM, 1), lambda i: (0, i, 0)),
        pl.BlockSpec((BM, F), lambda i: (i, 0)),
        pl.BlockSpec((F, D), lambda i: (0, 0)),
    ],
    out_specs=[
        pl.BlockSpec((BM, 1), lambda i: (i, 0)),
        pl.BlockSpec((BM, D), lambda i: (i, 0)),
    ],
    out_shape=[
        jax.ShapeDtypeStruct((N_PAD, 1), jnp.float32),
        jax.ShapeDtypeStruct((N_PAD, D), jnp.float32),
    ],
)


def _tc2_body(p_ref, src1_ref, dis_ref, b1_ref, src2_ref):
    t = p_ref[0] + p_ref[1] + src1_ref[...]
    h = jnp.maximum(dis_ref[...] * t + b1_ref[...], 0.0)
    src2_ref[...] = dis_ref[...] * h


_tc2 = pl.pallas_call(
    _tc2_body,
    grid=(_GRID,),
    in_specs=[
        pl.BlockSpec((2, BM, D), lambda i: (0, i, 0)),
        pl.BlockSpec((BM, D), lambda i: (i, 0)),
        pl.BlockSpec((BM, 1), lambda i: (i, 0)),
        pl.BlockSpec((1, D), lambda i: (0, 0)),
    ],
    out_specs=pl.BlockSpec((BM, D), lambda i: (i, 0)),
    out_shape=jax.ShapeDtypeStruct((N_PAD, D), jnp.float32),
)


def _tc3_body(q_ref, src2_ref, dis_ref, w2_ref, b2_ref, out_ref):
    t = dis_ref[...] * (q_ref[0] + q_ref[1] + src2_ref[...])
    out_ref[...] = jnp.dot(t, w2_ref[...],
                           preferred_element_type=jnp.float32) + b2_ref[...]


_tc3 = pl.pallas_call(
    _tc3_body,
    grid=(_GRID,),
    in_specs=[
        pl.BlockSpec((2, BM, D), lambda i: (0, i, 0)),
        pl.BlockSpec((BM, D), lambda i: (i, 0)),
        pl.BlockSpec((BM, 1), lambda i: (i, 0)),
        pl.BlockSpec((D, F), lambda i: (0, 0)),
        pl.BlockSpec((1, F), lambda i: (0, 0)),
    ],
    out_specs=pl.BlockSpec((BM, F), lambda i: (i, 0)),
    out_shape=jax.ShapeDtypeStruct((N_PAD, F), jnp.float32),
)


def kernel(x, edge_index, edge_weight, W1, b1, W2, b2):
    row = edge_index[0].astype(jnp.int32)
    col = edge_index[1].astype(jnp.int32)
    ew = edge_weight.astype(jnp.float32)
    e_tot = NS * (EPW_F + EPW_S)

    def _slabs(a):
        ap = jnp.pad(a, (0, e_tot - E))
        fast = ap[:NS * EPW_F].reshape(NS, NB, B)
        slow = jnp.pad(ap[NS * EPW_F:].reshape(NS, NCHS * CHB, B),
                       ((0, 0), (0, (NCHF - NCHS) * CHB), (0, 0)))
        parts = [slow, fast] if SLOW_C == 0 else [fast, slow]
        return jnp.concatenate(parts, axis=0)

    rowp = _slabs(row)
    colp = _slabs(col)
    ewp = _slabs(ew)
    dcol = jnp.pad(col, (0, e_tot - E)).reshape(NW, NB_DEG, B)
    dew = jnp.pad(ew, (0, e_tot - E)).reshape(NW, NB_DEG, B)
    xpad = jnp.pad(x, ((0, N_PAD - N), (0, 0)))

    degp = _deg_kernel(dcol, dew)
    dis, src1 = _tc1(degp.reshape(NC, N_PAD, 1), xpad, W1)
    p = _agg_kernel(src1, rowp, colp, ewp)
    src2 = _tc2(p, src1, dis, b1.reshape(1, D))
    q = _agg_kernel(src2, rowp, colp, ewp)
    out = _tc3(q, src2, dis, W2, b2.reshape(1, F))
    return out[:N]
